# Initial kernel scaffold; baseline (speedup 1.0000x reference)
import jax
import jax.numpy as jnp
from jax.experimental import pallas as pl


def kernel(x, edge_index, m_ids, edge_index_c, W1, b1, Ws1, Wn1, Ws2, Wn2, W_mu, W_lv, W_dec, Ws3, Wn3, Ws4, Wn4, W_out, b_out):
    raise NotImplementedError("write your pallas kernel here")


# speedup vs baseline: 1.9720x; 1.9720x over previous
import functools

import jax
import jax.numpy as jnp
from jax import lax
from jax.experimental import pallas as pl
from jax.experimental.pallas import tpu as pltpu
from jax.experimental.pallas import tpu_sc as plsc

CHUNK = 128
NSUB = 16
F32 = jnp.float32


def _mesh():
    return plsc.VectorSubcoreMesh(core_axis_name="c", subcore_axis_name="s")


def _make_segsum(NT, NACC, EP):
    nchunks = EP // CHUNK
    SR = NACC // NSUB
    assert NACC % NSUB == 0 and EP % CHUNK == 0

    def body(tlo, thi, src_r, dst_r, z_r, aglo, aghi,
             acc, idx_s, idx_d, rows, sem):
        c = lax.axis_index("c")
        s = lax.axis_index("s")
        pltpu.sync_copy(z_r.at[pl.ds(0, SR)], acc.at[pl.ds(s * SR, SR)])
        plsc.subcore_barrier()

        def run(t_r):
            def step(k, carry):
                ci = s + k * NSUB
                base = ci * CHUNK
                pltpu.sync_copy(src_r.at[pl.ds(base, CHUNK)], idx_s)
                pltpu.sync_copy(dst_r.at[pl.ds(base, CHUNK)], idx_d)
                pltpu.async_copy(t_r.at[idx_s], rows, sem).wait()
                pltpu.sync_copy(rows, acc.at[idx_d], add=True)
                return carry
            nk = (nchunks - s + NSUB - 1) // NSUB
            lax.fori_loop(0, nk, step, 0)

        @pl.when(c == 0)
        def _():
            run(tlo)

        @pl.when(c == 1)
        def _():
            run(thi)

        plsc.subcore_barrier()

        @pl.when(c == 0)
        def _():
            pltpu.sync_copy(acc.at[pl.ds(s * SR, SR)], aglo.at[pl.ds(s * SR, SR)])

        @pl.when(c == 1)
        def _():
            pltpu.sync_copy(acc.at[pl.ds(s * SR, SR)], aghi.at[pl.ds(s * SR, SR)])

    return pl.kernel(
        body,
        out_type=(jax.ShapeDtypeStruct((NACC, 128), F32),
                  jax.ShapeDtypeStruct((NACC, 128), F32)),
        mesh=_mesh(),
        scratch_types=[
            pltpu.VMEM_SHARED((NACC, 128), F32),
            pltpu.VMEM((CHUNK,), jnp.int32),
            pltpu.VMEM((CHUNK,), jnp.int32),
            pltpu.VMEM((CHUNK, 128), F32),
            pltpu.SemaphoreType.DMA,
        ],
    )


def _make_pool_gather(NT, M):
    nchunks = M // CHUNK

    def body(s1_r, alo_r, ahi_r, mid_r, gs, glo, ghi,
             idx, rows_w, rows_n, sem):
        c = lax.axis_index("c")
        s = lax.axis_index("s")
        wid = s * 2 + c

        def step(k, carry):
            ci = wid + k * 32
            base = ci * CHUNK
            pltpu.sync_copy(mid_r.at[pl.ds(base, CHUNK)], idx)
            pltpu.async_copy(s1_r.at[idx], rows_w, sem).wait()
            pltpu.sync_copy(rows_w, gs.at[pl.ds(base, CHUNK)])
            pltpu.async_copy(alo_r.at[idx], rows_n, sem).wait()
            pltpu.sync_copy(rows_n, glo.at[pl.ds(base, CHUNK)])
            pltpu.async_copy(ahi_r.at[idx], rows_n, sem).wait()
            pltpu.sync_copy(rows_n, ghi.at[pl.ds(base, CHUNK)])
            return carry

        nk = (nchunks - wid + 31) // 32
        lax.fori_loop(0, nk, step, 0)

    return pl.kernel(
        body,
        out_type=(jax.ShapeDtypeStruct((M, 256), F32),
                  jax.ShapeDtypeStruct((M, 128), F32),
                  jax.ShapeDtypeStruct((M, 128), F32)),
        mesh=_mesh(),
        scratch_types=[
            pltpu.VMEM((CHUNK,), jnp.int32),
            pltpu.VMEM((CHUNK, 256), F32),
            pltpu.VMEM((CHUNK, 128), F32),
            pltpu.SemaphoreType.DMA,
        ],
    )


def _make_unpool_scatter(M, NPAD):
    nchunks = M // CHUNK
    SR = NPAD // NSUB
    assert NPAD % NSUB == 0

    def body(us_lo, us_hi, ut_lo, ut_hi, sidx_r, z_r,
             s4lo, s4hi, t4lo, t4hi,
             idx, rows, sem):
        c = lax.axis_index("c")
        s = lax.axis_index("s")

        def zero_fill(o_ref):
            pltpu.sync_copy(z_r.at[pl.ds(0, SR)], o_ref.at[pl.ds(s * SR, SR)])

        def scatter(u_ref, o_ref):
            def step(k, carry):
                ci = s + k * NSUB
                base = ci * CHUNK
                pltpu.sync_copy(sidx_r.at[pl.ds(base, CHUNK)], idx.at[0])
                pltpu.sync_copy(u_ref.at[pl.ds(base, CHUNK)], rows)
                pltpu.async_copy(rows, o_ref.at[idx.at[0]], sem).wait()
                return carry
            nk = (nchunks - s + NSUB - 1) // NSUB
            lax.fori_loop(0, nk, step, 0)

        @pl.when(c == 0)
        def _():
            zero_fill(s4lo)
            zero_fill(t4lo)

        @pl.when(c == 1)
        def _():
            zero_fill(s4hi)
            zero_fill(t4hi)

        plsc.subcore_barrier()

        @pl.when(c == 0)
        def _():
            scatter(us_lo, s4lo)
            scatter(ut_lo, t4lo)

        @pl.when(c == 1)
        def _():
            scatter(us_hi, s4hi)
            scatter(ut_hi, t4hi)

    return pl.kernel(
        body,
        out_type=(jax.ShapeDtypeStruct((NPAD, 128), F32),
                  jax.ShapeDtypeStruct((NPAD, 128), F32),
                  jax.ShapeDtypeStruct((NPAD, 128), F32),
                  jax.ShapeDtypeStruct((NPAD, 128), F32)),
        mesh=_mesh(),
        scratch_types=[
            pltpu.VMEM((1, CHUNK), jnp.int32),
            pltpu.VMEM((CHUNK, 128), F32),
            pltpu.SemaphoreType.DMA,
        ],
    )


def _dot(a, b):
    return jnp.dot(a, b, preferred_element_type=F32)


def _enc_in(x, W1, b1, Ws1, Wn1, BM=1000):
    N, D = x.shape
    H = W1.shape[1]

    def body(x_r, W1_r, b1_r, Ws1_r, Wn1_r, s1_r, tlo_r, thi_r):
        h0 = jnp.maximum(_dot(x_r[...], W1_r[...]) + b1_r[...], 0.0)
        s1_r[...] = _dot(h0, Ws1_r[...])
        t1 = _dot(h0, Wn1_r[...])
        tlo_r[...] = t1[:, :128]
        thi_r[...] = t1[:, 128:]

    return pl.pallas_call(
        body,
        grid=(N // BM,),
        in_specs=[
            pl.BlockSpec((BM, D), lambda i: (i, 0)),
            pl.BlockSpec((D, H), lambda i: (0, 0)),
            pl.BlockSpec((1, H), lambda i: (0, 0)),
            pl.BlockSpec((H, H), lambda i: (0, 0)),
            pl.BlockSpec((H, H), lambda i: (0, 0)),
        ],
        out_specs=[
            pl.BlockSpec((BM, H), lambda i: (i, 0)),
            pl.BlockSpec((BM, 128), lambda i: (i, 0)),
            pl.BlockSpec((BM, 128), lambda i: (i, 0)),
        ],
        out_shape=[
            jax.ShapeDtypeStruct((N, H), F32),
            jax.ShapeDtypeStruct((N, 128), F32),
            jax.ShapeDtypeStruct((N, 128), F32),
        ],
    )(x, W1, b1.reshape(1, H), Ws1, Wn1)


def _coarse_mpl(g_s, g_lo, g_hi, Ws, Wn, NC, BM=1000):
    H = Ws.shape[0]

    def body(gs_r, glo_r, ghi_r, Ws_r, Wn_r, s_r, tlo_r, thi_r):
        hp = jnp.maximum(
            gs_r[...] + jnp.concatenate([glo_r[...], ghi_r[...]], axis=1), 0.0)
        s_r[...] = _dot(hp, Ws_r[...])
        t = _dot(hp, Wn_r[...])
        tlo_r[...] = t[:, :128]
        thi_r[...] = t[:, 128:]

    return pl.pallas_call(
        body,
        grid=(NC // BM,),
        in_specs=[
            pl.BlockSpec((BM, H), lambda i: (i, 0)),
            pl.BlockSpec((BM, 128), lambda i: (i, 0)),
            pl.BlockSpec((BM, 128), lambda i: (i, 0)),
            pl.BlockSpec((H, H), lambda i: (0, 0)),
            pl.BlockSpec((H, H), lambda i: (0, 0)),
        ],
        out_specs=[
            pl.BlockSpec((BM, H), lambda i: (i, 0)),
            pl.BlockSpec((BM, 128), lambda i: (i, 0)),
            pl.BlockSpec((BM, 128), lambda i: (i, 0)),
        ],
        out_shape=[
            jax.ShapeDtypeStruct((NC, H), F32),
            jax.ShapeDtypeStruct((NC, 128), F32),
            jax.ShapeDtypeStruct((NC, 128), F32),
        ],
    )(g_s, g_lo, g_hi, Ws, Wn)


def _latent(s2, a_lo, a_hi, W_mu, W_lv, W_dec, Ws3, Wn3, NC, BM=1000):
    H = Ws3.shape[0]
    L = W_mu.shape[1]
    nb = NC // BM
    denom = float(NC * L)

    def body(s2_r, alo_r, ahi_r, Wmu_r, Wlv_r, Wdec_r, Ws3_r, Wn3_r,
             s3_r, tlo_r, thi_r, kl_r, acc_r):
        i = pl.program_id(0)
        h2 = jnp.maximum(
            s2_r[...] + jnp.concatenate([alo_r[...], ahi_r[...]], axis=1), 0.0)
        mu = _dot(h2, Wmu_r[...])
        lv = _dot(h2, Wlv_r[...])
        part = jnp.sum(1.0 + lv - mu * mu - jnp.exp(lv))

        @pl.when(i == 0)
        def _():
            acc_r[0, 0] = 0.0

        acc_r[0, 0] += part
        kl_r[...] = jnp.reshape(-0.5 * acc_r[0, 0] / denom, (1, 1))

        hd0 = jnp.maximum(_dot(mu, Wdec_r[...]), 0.0)
        s3_r[...] = _dot(hd0, Ws3_r[...])
        t3 = _dot(hd0, Wn3_r[...])
        tlo_r[...] = t3[:, :128]
        thi_r[...] = t3[:, 128:]

    return pl.pallas_call(
        body,
        grid=(nb,),
        in_specs=[
            pl.BlockSpec((BM, H), lambda i: (i, 0)),
            pl.BlockSpec((BM, 128), lambda i: (i, 0)),
            pl.BlockSpec((BM, 128), lambda i: (i, 0)),
            pl.BlockSpec((H, L), lambda i: (0, 0)),
            pl.BlockSpec((H, L), lambda i: (0, 0)),
            pl.BlockSpec((L, H), lambda i: (0, 0)),
            pl.BlockSpec((H, H), lambda i: (0, 0)),
            pl.BlockSpec((H, H), lambda i: (0, 0)),
        ],
        out_specs=[
            pl.BlockSpec((BM, H), lambda i: (i, 0)),
            pl.BlockSpec((BM, 128), lambda i: (i, 0)),
            pl.BlockSpec((BM, 128), lambda i: (i, 0)),
            pl.BlockSpec((1, 1), lambda i: (0, 0)),
        ],
        out_shape=[
            jax.ShapeDtypeStruct((NC, H), F32),
            jax.ShapeDtypeStruct((NC, 128), F32),
            jax.ShapeDtypeStruct((NC, 128), F32),
            jax.ShapeDtypeStruct((1, 1), F32),
        ],
        scratch_shapes=[pltpu.SMEM((1, 1), F32)],
    )(s2, a_lo, a_hi, W_mu, W_lv, W_dec, Ws3, Wn3)


def _dec_mid(s3, a_lo, a_hi, Ws4, Wn4, NC, MPAD, BM=1000):
    H = Ws4.shape[0]

    def body(s3_r, alo_r, ahi_r, Ws4_r, Wn4_r, uslo_r, ushi_r, utlo_r, uthi_r):
        hd1 = jnp.maximum(
            s3_r[...] + jnp.concatenate([alo_r[...], ahi_r[...]], axis=1), 0.0)
        us = _dot(hd1, Ws4_r[...])
        ut = _dot(hd1, Wn4_r[...])
        uslo_r[...] = us[:, :128]
        ushi_r[...] = us[:, 128:]
        utlo_r[...] = ut[:, :128]
        uthi_r[...] = ut[:, 128:]

    return pl.pallas_call(
        body,
        grid=(NC // BM,),
        in_specs=[
            pl.BlockSpec((BM, H), lambda i: (i, 0)),
            pl.BlockSpec((BM, 128), lambda i: (i, 0)),
            pl.BlockSpec((BM, 128), lambda i: (i, 0)),
            pl.BlockSpec((H, H), lambda i: (0, 0)),
            pl.BlockSpec((H, H), lambda i: (0, 0)),
        ],
        out_specs=[
            pl.BlockSpec((BM, 128), lambda i: (i, 0)),
            pl.BlockSpec((BM, 128), lambda i: (i, 0)),
            pl.BlockSpec((BM, 128), lambda i: (i, 0)),
            pl.BlockSpec((BM, 128), lambda i: (i, 0)),
        ],
        out_shape=[
            jax.ShapeDtypeStruct((MPAD, 128), F32),
            jax.ShapeDtypeStruct((MPAD, 128), F32),
            jax.ShapeDtypeStruct((MPAD, 128), F32),
            jax.ShapeDtypeStruct((MPAD, 128), F32),
        ],
    )(s3, a_lo, a_hi, Ws4, Wn4)


def _out_mlp(s4_lo, s4_hi, a_lo, a_hi, W_out, b_out, N, BM=1000):
    H = W_out.shape[0]
    D = W_out.shape[1]

    def body(slo_r, shi_r, alo_r, ahi_r, Wo_r, bo_r, o_r):
        full = jnp.maximum(
            jnp.concatenate([slo_r[...] + alo_r[...],
                             shi_r[...] + ahi_r[...]], axis=1), 0.0)
        o_r[...] = _dot(full, Wo_r[...]) + bo_r[...]

    return pl.pallas_call(
        body,
        grid=(N // BM,),
        in_specs=[
            pl.BlockSpec((BM, 128), lambda i: (i, 0)),
            pl.BlockSpec((BM, 128), lambda i: (i, 0)),
            pl.BlockSpec((BM, 128), lambda i: (i, 0)),
            pl.BlockSpec((BM, 128), lambda i: (i, 0)),
            pl.BlockSpec((H, D), lambda i: (0, 0)),
            pl.BlockSpec((1, D), lambda i: (0, 0)),
        ],
        out_specs=[pl.BlockSpec((BM, D), lambda i: (i, 0))],
        out_shape=[jax.ShapeDtypeStruct((N, D), F32)],
    )(s4_lo, s4_hi, a_lo, a_hi, W_out, b_out.reshape(1, D))[0]


def kernel(x, edge_index, m_ids, edge_index_c, W1, b1, Ws1, Wn1, Ws2, Wn2,
           W_mu, W_lv, W_dec, Ws3, Wn3, Ws4, Wn4, W_out, b_out):
    N, D = x.shape
    H = W1.shape[1]
    NC = m_ids.shape[0]
    E = edge_index.shape[1]
    EC = edge_index_c.shape[1]

    def _acc_rows(min_rows):
        per = (min_rows + NSUB - 1) // NSUB
        return NSUB * ((per + 7) // 8 * 8)

    src = edge_index[0]
    dst = edge_index[1]
    EP = (E + CHUNK - 1) // CHUNK * CHUNK
    NACC = _acc_rows(N + (0 if EP == E else 1))
    if EP != E:
        src = jnp.concatenate([src, jnp.zeros((EP - E,), jnp.int32)])
        dst = jnp.concatenate([dst, jnp.full((EP - E,), N, jnp.int32)])

    src_c = edge_index_c[0]
    dst_c = edge_index_c[1]
    ECP = (EC + CHUNK - 1) // CHUNK * CHUNK
    NCACC = _acc_rows(NC + (0 if ECP == EC else 1))
    if ECP != EC:
        src_c = jnp.concatenate([src_c, jnp.zeros((ECP - EC,), jnp.int32)])
        dst_c = jnp.concatenate([dst_c, jnp.full((ECP - EC,), NC, jnp.int32)])

    MPAD = (NC + CHUNK - 1) // CHUNK * CHUNK
    mid_pad = jnp.concatenate([m_ids, jnp.zeros((MPAD - NC,), jnp.int32)]) \
        if MPAD != NC else m_ids

    NPAD = _acc_rows(N + 1)
    last = jnp.concatenate([m_ids[1:] != m_ids[:-1],
                            jnp.ones((1,), dtype=bool)])
    sidx = jnp.where(last, m_ids, N)
    sidx_pad = jnp.concatenate([sidx, jnp.full((MPAD - NC,), N, jnp.int32)]) \
        if MPAD != NC else sidx

    NACC4 = _acc_rows(N)
    zrows = max(NACC // NSUB, NCACC // NSUB, NPAD // NSUB, NACC4 // NSUB)
    zeros = jnp.zeros((zrows, 128), F32)

    s1, t1_lo, t1_hi = _enc_in(x, W1, b1, Ws1, Wn1)
    a1_lo, a1_hi = _make_segsum(N, NACC, EP)(t1_lo, t1_hi, src, dst, zeros)
    g_s, g_lo, g_hi = _make_pool_gather(N, MPAD)(s1, a1_lo, a1_hi, mid_pad)
    s2, t2_lo, t2_hi = _coarse_mpl(g_s, g_lo, g_hi, Ws2, Wn2, NC)
    a2_lo, a2_hi = _make_segsum(NC, NCACC, ECP)(t2_lo, t2_hi, src_c, dst_c, zeros)

    s3, t3_lo, t3_hi, kl_arr = _latent(s2, a2_lo, a2_hi, W_mu, W_lv, W_dec,
                                       Ws3, Wn3, NC)

    a3_lo, a3_hi = _make_segsum(NC, NCACC, ECP)(t3_lo, t3_hi, src_c, dst_c, zeros)
    us_lo, us_hi, ut_lo, ut_hi = _dec_mid(s3, a3_lo, a3_hi, Ws4, Wn4, NC, MPAD)
    s4_lo, s4_hi, t4_lo, t4_hi = _make_unpool_scatter(MPAD, NPAD)(
        us_lo, us_hi, ut_lo, ut_hi, sidx_pad, zeros)
    a4_lo, a4_hi = _make_segsum(NPAD, NACC4, EP)(t4_lo, t4_hi, src, dst, zeros)
    out = _out_mlp(s4_lo, s4_hi, a4_lo, a4_hi, W_out, b_out, N)

    return (out, kl_arr[0, 0])
